# baseline (speedup 1.0000x reference)
import jax
import jax.numpy as jnp
from jax import lax
from jax.experimental import pallas as pl
from jax.experimental.pallas import tpu as pltpu


def _vq_kernel(x_ref, emb_ref, quant_ref, counts_ref, sse_ref):
    b = pl.program_id(1)

    @pl.when(b == 0)
    def _():
        counts_ref[...] = jnp.zeros_like(counts_ref)
        sse_ref[...] = jnp.zeros_like(sse_ref)

    x = x_ref[0, 0]
    e = emb_ref[0]
    M = e.shape[0]
    HW = x.shape[1]

    e2 = jnp.sum(e * e, axis=1, keepdims=True)
    scores = e2 + jnp.dot(e * (-2.0), x,
                          preferred_element_type=jnp.float32)

    min_s = jnp.min(scores, axis=0, keepdims=True)
    iota_m = lax.broadcasted_iota(jnp.int32, scores.shape, 0)
    idx = jnp.min(jnp.where(scores == min_s, iota_m, jnp.int32(M)),
                  axis=0, keepdims=True)
    enc_t = (iota_m == idx).astype(jnp.float32)

    quant_ref[0, 0] = lax.dot_general(
        e, enc_t, (((0,), (0,)), ((), ())),
        preferred_element_type=jnp.float32)

    ones_col = jnp.ones((HW, 1), jnp.float32)
    counts_ref[0] += jnp.dot(enc_t, ones_col,
                             preferred_element_type=jnp.float32)

    x2 = jnp.sum(x * x, axis=0, keepdims=True)
    sse_ref[0] += jnp.sum(jnp.maximum(x2 + min_s, 0.0),
                          axis=1, keepdims=True)


def _vq_forward(x, embedding, commitment_cost=0.25):
    B, C, H, W = x.shape
    N, M, D = embedding.shape
    assert C == N * D
    HW = H * W
    L = B * HW

    x_v = x.reshape(B, N, D, HW)

    out_shapes = (
        jax.ShapeDtypeStruct((B, N, D, HW), jnp.float32),
        jax.ShapeDtypeStruct((N, M, 1), jnp.float32),
        jax.ShapeDtypeStruct((N, 1, 1), jnp.float32),
    )

    quant, counts, sse = pl.pallas_call(
        _vq_kernel,
        out_shape=out_shapes,
        grid_spec=pltpu.PrefetchScalarGridSpec(
            num_scalar_prefetch=0,
            grid=(N, B),
            in_specs=[
                pl.BlockSpec((1, 1, D, HW), lambda n, b: (b, n, 0, 0)),
                pl.BlockSpec((1, M, D), lambda n, b: (n, 0, 0)),
            ],
            out_specs=[
                pl.BlockSpec((1, 1, D, HW), lambda n, b: (b, n, 0, 0)),
                pl.BlockSpec((1, M, 1), lambda n, b: (n, 0, 0)),
                pl.BlockSpec((1, 1, 1), lambda n, b: (n, 0, 0)),
            ],
        ),
        compiler_params=pltpu.CompilerParams(
            dimension_semantics=("parallel", "arbitrary")),
    )(x_v, embedding)

    loss = commitment_cost * (jnp.sum(sse) / (N * L * D))

    out = quant.reshape(B, C, H, W)

    avg_probs = counts[:, :, 0] / L
    perplexity = jnp.exp(-jnp.sum(avg_probs * jnp.log(avg_probs + 1e-10),
                                  axis=-1))
    return out, loss, jnp.sum(perplexity)


def kernel(x, embedding):
    return _vq_forward(x, embedding, commitment_cost=0.25)
